# baseline (speedup 1.0000x reference)
import functools

import numpy as np

import jax
import jax.numpy as jnp
from jax import lax
from jax.experimental import pallas as pl
from jax.experimental.pallas import tpu as pltpu
from jax.experimental.pallas import tpu_sc as plsc

N_ATOMS_K = 10000
N_PAIRS_K = 160000
XDIM = 3
XPAD = 4
N_PROP_K = 128

TC_BLK = 1000
TC_GRID = N_ATOMS_K // TC_BLK

L = 16


def _proj_body(px_ref, wi_ref, wj_ref, w0_ref, w1_ref, yi_ref, yj_ref,
               a_scr, b_scr):
    @pl.when(pl.program_id(0) == 0)
    def _():
        w01 = jnp.dot(w0_ref[...], w1_ref[...],
                      preferred_element_type=jnp.float32,
                      precision=lax.Precision.HIGHEST)
        a_scr[...] = jnp.dot(wi_ref[...], w01,
                             preferred_element_type=jnp.float32,
                             precision=lax.Precision.HIGHEST)
        b_scr[...] = jnp.dot(wj_ref[...], w01,
                             preferred_element_type=jnp.float32,
                             precision=lax.Precision.HIGHEST)

    x = px_ref[...].reshape(TC_BLK * XDIM, N_PROP_K)

    def proj(w):
        y = jnp.dot(x, w, preferred_element_type=jnp.float32,
                    precision=lax.Precision.HIGHEST)
        y = y.astype(jnp.bfloat16).reshape(TC_BLK, XDIM, N_PROP_K)
        return jnp.concatenate([y, y[:, XDIM - 1:, :]], axis=1)

    yi_ref[...] = proj(a_scr[...])
    yj_ref[...] = proj(b_scr[...])


@jax.jit
def _project(px, Wi, Wj, W0, W1):
    wspec = pl.BlockSpec((N_PROP_K, N_PROP_K), lambda i: (0, 0))
    return pl.pallas_call(
        _proj_body,
        grid=(TC_GRID,),
        in_specs=[
            pl.BlockSpec((TC_BLK, XDIM, N_PROP_K), lambda i: (i, 0, 0)),
            wspec, wspec, wspec, wspec,
        ],
        out_specs=[pl.BlockSpec((TC_BLK, XPAD, N_PROP_K),
                                lambda i: (i, 0, 0))] * 2,
        out_shape=[jax.ShapeDtypeStruct((N_ATOMS_K, XPAD, N_PROP_K),
                                        jnp.bfloat16)] * 2,
        scratch_shapes=[
            pltpu.VMEM((N_PROP_K, N_PROP_K), jnp.float32),
            pltpu.VMEM((N_PROP_K, N_PROP_K), jnp.float32),
        ],
    )(px, Wi, Wj, W0, W1)


def _make_sc_gather():
    info = plsc.get_sparse_core_info()
    nc, ns = info.num_cores, info.num_subcores
    nw = nc * ns
    per_w = N_PAIRS_K // nw
    chunk = 40
    n_real = per_w // chunk
    n_chunks = n_real + 1

    mesh = plsc.VectorSubcoreMesh(core_axis_name="c", subcore_axis_name="s")

    gbuf_t = pltpu.VMEM((chunk, XPAD // 2, N_PROP_K), jnp.int32)
    sbuf_t = pltpu.VMEM((chunk, XPAD, N_PROP_K), jnp.float32)

    @functools.partial(
        pl.kernel,
        mesh=mesh,
        out_type=jax.ShapeDtypeStruct((N_PAIRS_K, XDIM, N_PROP_K),
                                      jnp.float32),
        compiler_params=pltpu.CompilerParams(needs_layout_passes=False),
        scratch_types=[
            pltpu.VMEM((n_chunks, chunk), jnp.int32),
            pltpu.VMEM((n_chunks, chunk), jnp.int32),
            gbuf_t, gbuf_t, gbuf_t, gbuf_t, sbuf_t, sbuf_t,
            pltpu.SemaphoreType.DMA, pltpu.SemaphoreType.DMA,
            pltpu.SemaphoreType.DMA, pltpu.SemaphoreType.DMA,
            pltpu.SemaphoreType.DMA, pltpu.SemaphoreType.DMA,
        ],
    )
    def sc_gather(yi_hbm, yj_hbm, idxi_hbm, idxj_hbm, out_hbm,
                  idxi_v, idxj_v, ga0, ga1, gb0, gb1, st0, st1,
                  gsa0, gsa1, gsb0, gsb1, sts0, sts1):
        wid = lax.axis_index("s") * nc + lax.axis_index("c")
        base = wid * per_w
        ga = (ga0, ga1)
        gb = (gb0, gb1)
        st = (st0, st1)
        gsa = (gsa0, gsa1)
        gsb = (gsb0, gsb1)
        sts = (sts0, sts1)
        pltpu.sync_copy(idxi_hbm.at[wid], idxi_v)
        pltpu.sync_copy(idxj_hbm.at[wid], idxj_v)

        def issue_gather(c, par):
            pltpu.async_copy(yi_hbm.at[idxi_v.at[c]], ga[par], gsa[par])
            pltpu.async_copy(yj_hbm.at[idxj_v.at[c]], gb[par], gsb[par])

        def wait_gather(c, par):
            pltpu.make_async_copy(
                yi_hbm.at[idxi_v.at[c]], ga[par], gsa[par]).wait()
            pltpu.make_async_copy(
                yj_hbm.at[idxj_v.at[c]], gb[par], gsb[par]).wait()

        def wait_store(c, par):
            pltpu.make_async_copy(
                st[par].at[:, pl.ds(0, XDIM)],
                out_hbm.at[pl.ds(base + c * chunk, chunk)],
                sts[par]).wait()

        issue_gather(0, 0)

        def step(s, carry):
            for b in range(2):
                c = 2 * s + b
                par = b
                opar = 1 - b

                @pl.when(s >= 1)
                def _():
                    wait_store(c - 2, par)

                @pl.when(c + 1 <= n_chunks - 1)
                def _():
                    issue_gather(c + 1, opar)

                wait_gather(c, par)

                @pl.when(c <= n_real - 1)
                def _():
                    def row_body(r, cr):
                        for x in range(XDIM):
                            for k in range(N_PROP_K // (2 * L)):
                                flat = x * (N_PROP_K // 2) + k * L
                                u, v = divmod(flat, N_PROP_K)
                                pa = ga[par][r, u, pl.ds(v, L)]
                                pb = gb[par][r, u, pl.ds(v, L)]
                                sab = (plsc.bitcast(pa, jnp.bfloat16)
                                       + plsc.bitcast(pb, jnp.bfloat16))
                                p = plsc.bitcast(sab, jnp.int32)
                                ev = plsc.bitcast(p << 16, jnp.float32)
                                od = plsc.bitcast(
                                    p & jnp.int32(-65536), jnp.float32)
                                col0 = k * 2 * L
                                st[par][r, x, pl.ds(col0, L)] = ev
                                st[par][r, x, pl.ds(col0 + L, L)] = od
                        return cr

                    lax.fori_loop(0, chunk, row_body, 0)
                    pltpu.async_copy(
                        st[par].at[:, pl.ds(0, XDIM)],
                        out_hbm.at[pl.ds(base + c * chunk, chunk)],
                        sts[par])
            return carry

        lax.fori_loop(0, n_chunks // 2, step, 0)
        wait_store(n_real - 1, 0)

    return sc_gather, nw, per_w, n_chunks, chunk


def _lane_perm():
    perm = np.empty(N_PROP_K, np.int32)
    for k in range(N_PROP_K // 32):
        for i in range(16):
            perm[k * 32 + 2 * i] = k * 32 + i
            perm[k * 32 + 2 * i + 1] = k * 32 + 16 + i
    return perm


def kernel(ind_2, px, Wi, Wj, W0, W1):
    sc_gather, nw, per_w, n_chunks, chunk = _make_sc_gather()

    w1p = jnp.take(W1, jnp.asarray(_lane_perm()), axis=1)
    yi, yj = _project(px, Wi, Wj, W0, w1p)

    def as_i32(t):
        t = jax.lax.bitcast_convert_type(
            t.reshape(N_ATOMS_K, XPAD, N_PROP_K // 2, 2), jnp.int32)
        return t.reshape(N_ATOMS_K, XPAD // 2, N_PROP_K)

    yi = as_i32(yi)
    yj = as_i32(yj)

    ind = ind_2.astype(jnp.int32)
    pad = n_chunks * chunk - per_w

    def prep(col):
        a = col.reshape(nw, per_w)
        a = jnp.pad(a, ((0, 0), (0, pad)))
        return a.reshape(nw, n_chunks, chunk)

    idxi = prep(ind[:, 0])
    idxj = prep(ind[:, 1])

    return sc_gather(yi, yj, idxi, idxj)
